# baseline (speedup 1.0000x reference)
import functools

import jax
import jax.numpy as jnp
from jax import lax
from jax.experimental import pallas as pl
from jax.experimental.pallas import tpu as pltpu
from jax.experimental.pallas import tpu_sc as plsc

VOCAB = 100000
DIM = 128
B = 16384
CTX = 50
NNEG = 50
NSLOT = 64

NC = 2
NS = 16
NW = NC * NS
BPW = B // NW
CB = 8
NCHUNK = BPW // CB
NIDX = CB * CTX
GSZ = 2
NGRP = NCHUNK // GSZ
NL = 16
DIMW = DIM // 2
NWRD = DIMW // NL
_HIMASK = jnp.int32(-65536)

_GATHER_DNUMS = lax.GatherDimensionNumbers(
    offset_dims=(), collapsed_slice_dims=(0,), start_index_map=(0,))

_ILV = plsc.PackFormat.INTERLEAVED


def _lane_gather(v, idx):
    return lax.gather(v, idx[:, None], _GATHER_DNUMS, (1,),
                      mode=lax.GatherScatterMode.PROMISE_IN_BOUNDS)


def _allsum(v, lanes):
    for sh in (8, 4, 2, 1):
        v = v + _lane_gather(v, lanes ^ sh)
    return v


def _reduce16(vs, lanes, masks):
    sh = 1
    for m in masks:
        half = []
        for i in range(len(vs) // 2):
            a, b = vs[2 * i], vs[2 * i + 1]
            half.append(jnp.where(m, a + _lane_gather(a, lanes ^ sh),
                                  b + _lane_gather(b, lanes ^ sh)))
        vs = half
        sh *= 2
    return vs[0]


def _sc_body(ctx_idx_hbm, tgt_idx_hbm, neg_idx_hbm, emb_in_hbm, emb_out_hbm,
             ts_out, ns_out,
             ctx_idx_v, neg_idx_v, tgt_idx_v,
             ctx_rows, neg_rows, tgt_rows,
             ns_gbuf, ts_gbuf, sem_c, sem_n, sem_t, sem_i, sem_o):
    cid = lax.axis_index("c")
    sid = lax.axis_index("s")
    wid = sid * NC + cid
    wbase = wid * BPW
    lanes = lax.iota(jnp.int32, NL)
    masks = [(lanes & sh) == 0 for sh in (1, 2, 4, 8)]

    def stage_idx_descs(slot, c):
        base_b = wbase + c * CB
        return (
            pltpu.make_async_copy(
                ctx_idx_hbm.at[pl.ds(base_b * CTX, NIDX)],
                ctx_idx_v.at[pl.ds(slot * NIDX, NIDX)], sem_i),
            pltpu.make_async_copy(
                neg_idx_hbm.at[pl.ds(base_b * NNEG, NIDX)],
                neg_idx_v.at[pl.ds(slot * NIDX, NIDX)], sem_i),
        )

    def rows_descs(slot):
        return (
            pltpu.make_async_copy(
                emb_in_hbm.at[ctx_idx_v.at[pl.ds(slot * NIDX, NIDX)]],
                ctx_rows.at[slot], sem_c),
            pltpu.make_async_copy(
                emb_out_hbm.at[neg_idx_v.at[pl.ds(slot * NIDX, NIDX)]],
                neg_rows.at[slot], sem_n),
        )

    def tgt_desc(slot):
        return pltpu.make_async_copy(
            emb_out_hbm.at[tgt_idx_v.at[pl.ds(slot * NL, NL)]],
            tgt_rows.at[slot], sem_t)

    def out_descs(oslot, g):
        return (
            pltpu.make_async_copy(
                ns_gbuf.at[oslot],
                ns_out.at[pl.ds(wbase + g * NL, NL), :], sem_o),
            pltpu.make_async_copy(
                ts_gbuf.at[pl.ds(oslot * NL, NL)],
                ts_out.at[pl.ds(wbase + g * NL, NL)], sem_o),
        )

    pltpu.sync_copy(ctx_idx_hbm.at[pl.ds(wbase * CTX, NIDX)],
                    ctx_idx_v.at[pl.ds(0, NIDX)])
    pltpu.sync_copy(neg_idx_hbm.at[pl.ds(wbase * NNEG, NIDX)],
                    neg_idx_v.at[pl.ds(0, NIDX)])
    pltpu.sync_copy(tgt_idx_hbm.at[pl.ds(wbase, NL)],
                    tgt_idx_v.at[pl.ds(0, NL)])
    for d in rows_descs(0):
        d.start()
    tgt_desc(0).start()
    for d in stage_idx_descs(1, 1):
        d.start()

    def unpack_row(rows, slot, r):
        out = []
        for w in range(NWRD):
            x = rows[slot, r, pl.ds(NL * w, NL)]
            lo = plsc.bitcast(x << 16, jnp.float32)
            hi = plsc.bitcast(x & _HIMASK, jnp.float32)
            out.append((lo, hi))
        return out

    def chunk_body(c, ts_vec):
        cur = c % 2
        nxt = 1 - cur
        g = c // GSZ
        u = c % GSZ
        tslot = g % 2
        oslot = tslot

        for d in rows_descs(cur):
            d.wait()

        @pl.when(u == 0)
        def _():
            tgt_desc(tslot).wait()

        @pl.when(c + 1 < NCHUNK)
        def _():
            for d in stage_idx_descs(nxt, c + 1):
                d.wait()
            for d in rows_descs(nxt):
                d.start()

        @pl.when((u == GSZ - 1) & (g + 1 < NGRP))
        def _():
            tgt_desc(1 - tslot).start()

        @pl.when(c + 2 < NCHUNK)
        def _():
            for d in stage_idx_descs(cur, c + 2):
                d.start()

        @pl.when((u == 0) & (g >= 2))
        def _():
            for d in out_descs(oslot, g - 2):
                d.wait()

        @pl.when((u == 0) & (g + 1 < NGRP))
        def _():
            pltpu.sync_copy(tgt_idx_hbm.at[pl.ds(wbase + (g + 1) * NL, NL)],
                            tgt_idx_v.at[pl.ds((1 - tslot) * NL, NL)])

        def b_body(b, ts_vec):
            rbase = b * CTX

            def j_body(j, acc):
                row = unpack_row(ctx_rows, cur, rbase + j)
                return tuple(acc[2 * w + h] + row[w][h]
                             for w in range(NWRD) for h in range(2))

            acc = lax.fori_loop(
                0, CTX, j_body,
                tuple(jnp.zeros((NL,), jnp.float32) for _ in range(2 * NWRD)))

            trow = u * CB + b
            tr = unpack_row(tgt_rows, tslot, trow)
            p = acc[0] * tr[0][0] + acc[1] * tr[0][1]
            for w in range(1, NWRD):
                p = p + acc[2 * w] * tr[w][0] + acc[2 * w + 1] * tr[w][1]
            ts_vec = jnp.where(lanes == trow, _allsum(p, lanes), ts_vec)

            def dot(r):
                row = unpack_row(neg_rows, cur, r)
                q = acc[0] * row[0][0] + acc[1] * row[0][1]
                for w in range(1, NWRD):
                    q = q + acc[2 * w] * row[w][0] + acc[2 * w + 1] * row[w][1]
                return q

            for gg in range(1):
                qs = [dot(rbase + gg * NL + k) for k in range(NL)]
                ns_gbuf[oslot, trow, pl.ds(gg * NL, NL)] = _reduce16(
                    qs, lanes, masks)

            return ts_vec

        ts_vec = lax.fori_loop(0, CB, b_body, ts_vec)

        @pl.when(u == GSZ - 1)
        def _():
            ts_gbuf[pl.ds(oslot * NL, NL)] = ts_vec
            for d in out_descs(oslot, g):
                d.start()

        return jnp.where(u == GSZ - 1, jnp.zeros((NL,), jnp.float32), ts_vec)

    lax.fori_loop(0, NCHUNK, chunk_body, jnp.zeros((NL,), jnp.float32))

    for gg in (NGRP - 2, NGRP - 1):
        for d in out_descs(gg % 2, gg):
            d.wait()


_sc_call = functools.partial(
    pl.kernel,
    mesh=plsc.VectorSubcoreMesh(core_axis_name="c", subcore_axis_name="s"),
    compiler_params=pltpu.CompilerParams(needs_layout_passes=False,
                                         use_tc_tiling_on_sc=False),
    out_type=[
        jax.ShapeDtypeStruct((B,), jnp.float32),
        jax.ShapeDtypeStruct((B, NSLOT), jnp.float32),
    ],
    scratch_types=[
        pltpu.VMEM((2 * NIDX,), jnp.int32),
        pltpu.VMEM((2 * NIDX,), jnp.int32),
        pltpu.VMEM((2 * NL,), jnp.int32),
        pltpu.VMEM((2, NIDX, DIMW), jnp.int32),
        pltpu.VMEM((2, NIDX, DIMW), jnp.int32),
        pltpu.VMEM((2, NL, DIMW), jnp.int32),
        pltpu.VMEM((2, NL, NSLOT), jnp.float32),
        pltpu.VMEM((2 * NL,), jnp.float32),
        pltpu.SemaphoreType.DMA,
        pltpu.SemaphoreType.DMA,
        pltpu.SemaphoreType.DMA,
        pltpu.SemaphoreType.DMA,
        pltpu.SemaphoreType.DMA,
    ],
)(_sc_body)


def _loss_body(ts_ref, ns_ref, out_ref):
    ts = ts_ref[...]
    ns = ns_ref[...]

    def softplus(x):
        return jnp.maximum(x, 0.0) + jnp.log1p(jnp.exp(-jnp.abs(x)))

    t_term = jnp.sum(softplus(-ts)) / B
    mask = lax.broadcasted_iota(jnp.int32, ns.shape, 1) < NNEG
    n_term = jnp.sum(jnp.where(mask, softplus(ns), 0.0)) / (B * NNEG)
    out_ref[0, 0] = t_term + n_term


_loss_call = pl.pallas_call(
    _loss_body,
    out_shape=jax.ShapeDtypeStruct((1, 1), jnp.float32),
    out_specs=pl.BlockSpec(memory_space=pltpu.SMEM),
)


def _pack_table(t):
    return lax.bitcast_convert_type(
        t.astype(jnp.bfloat16).reshape(VOCAB, DIM // 2, 2), jnp.int32)


def kernel(context, target, negative_samples, emb_in, emb_out):
    ctx_flat = context.reshape(-1)
    neg_flat = negative_samples.reshape(-1)
    ts, ns = _sc_call(ctx_flat, target, neg_flat,
                      _pack_table(emb_in), _pack_table(emb_out))
    loss = _loss_call(ts.reshape(B // DIM, DIM), ns)
    return loss[0, 0]


# speedup vs baseline: 2.7187x; 2.7187x over previous
import functools

import jax
import jax.numpy as jnp
from jax import lax
from jax.experimental import pallas as pl
from jax.experimental.pallas import tpu as pltpu
from jax.experimental.pallas import tpu_sc as plsc

VOCAB = 100000
DIM = 128
B = 16384
CTX = 50
NNEG = 50
NSLOT = 64

NC = 2
NS = 16
NW = NC * NS
BPW = B // NW
CB = 4
NCHUNK = BPW // CB
NIDX = CB * CTX
GSZ = 4
NGRP = NCHUNK // GSZ
NL = 16
ND = DIM // NL

_GATHER_DNUMS = lax.GatherDimensionNumbers(
    offset_dims=(), collapsed_slice_dims=(0,), start_index_map=(0,))


def _lane_gather(v, idx):
    return lax.gather(v, idx[:, None], _GATHER_DNUMS, (1,),
                      mode=lax.GatherScatterMode.PROMISE_IN_BOUNDS)


def _allsum(v, lanes):
    for sh in (8, 4, 2, 1):
        v = v + _lane_gather(v, lanes ^ sh)
    return v


def _reduce16(vs, lanes, masks):
    sh = 1
    for m in masks:
        half = []
        for i in range(len(vs) // 2):
            a, b = vs[2 * i], vs[2 * i + 1]
            half.append(jnp.where(m, a + _lane_gather(a, lanes ^ sh),
                                  b + _lane_gather(b, lanes ^ sh)))
        vs = half
        sh *= 2
    return vs[0]


def _sc_body(ctx_idx_hbm, tgt_idx_hbm, neg_idx_hbm, emb_in_hbm, emb_out_hbm,
             ts_out, ns_out,
             ctx_idx_v, neg_idx_v, tgt_idx_v,
             ctx_rows, neg_rows, tgt_rows,
             ns_gbuf, ts_gbuf, sem_c, sem_n, sem_t, sem_i, sem_o):
    cid = lax.axis_index("c")
    sid = lax.axis_index("s")
    wid = sid * NC + cid
    wbase = wid * BPW
    lanes = lax.iota(jnp.int32, NL)
    masks = [(lanes & sh) == 0 for sh in (1, 2, 4, 8)]

    def stage_idx_descs(slot, c):
        base_b = wbase + c * CB
        return (
            pltpu.make_async_copy(
                ctx_idx_hbm.at[pl.ds(base_b * CTX, NIDX)],
                ctx_idx_v.at[pl.ds(slot * NIDX, NIDX)], sem_i),
            pltpu.make_async_copy(
                neg_idx_hbm.at[pl.ds(base_b * NNEG, NIDX)],
                neg_idx_v.at[pl.ds(slot * NIDX, NIDX)], sem_i),
        )

    def rows_descs(slot):
        return (
            pltpu.make_async_copy(
                emb_in_hbm.at[ctx_idx_v.at[pl.ds(slot * NIDX, NIDX)]],
                ctx_rows.at[slot], sem_c),
            pltpu.make_async_copy(
                emb_out_hbm.at[neg_idx_v.at[pl.ds(slot * NIDX, NIDX)]],
                neg_rows.at[slot], sem_n),
        )

    def out_descs(oslot, g):
        return (
            pltpu.make_async_copy(
                ns_gbuf.at[oslot],
                ns_out.at[pl.ds(wbase + g * NL, NL), :], sem_o),
            pltpu.make_async_copy(
                ts_gbuf.at[pl.ds(oslot * NL, NL)],
                ts_out.at[pl.ds(wbase + g * NL, NL)], sem_o),
        )

    def tgt_desc(slot):
        return pltpu.make_async_copy(
            emb_out_hbm.at[tgt_idx_v.at[pl.ds(slot * NL, NL)]],
            tgt_rows.at[slot], sem_t)

    pltpu.sync_copy(ctx_idx_hbm.at[pl.ds(wbase * CTX, NIDX)],
                    ctx_idx_v.at[pl.ds(0, NIDX)])
    pltpu.sync_copy(neg_idx_hbm.at[pl.ds(wbase * NNEG, NIDX)],
                    neg_idx_v.at[pl.ds(0, NIDX)])
    pltpu.sync_copy(tgt_idx_hbm.at[pl.ds(wbase, NL)], tgt_idx_v.at[pl.ds(0, NL)])
    for d in rows_descs(0):
        d.start()
    tgt_desc(0).start()
    for d in stage_idx_descs(1, 1):
        d.start()

    def chunk_body(c, ts_vec):
        cur = c % 2
        nxt = 1 - cur
        g = c // GSZ
        u = c % GSZ
        tslot = g % 2
        oslot = tslot

        @pl.when(c + 1 < NCHUNK)
        def _():
            for d in stage_idx_descs(nxt, c + 1):
                d.wait()
            for d in rows_descs(nxt):
                d.start()

        @pl.when((u == GSZ - 1) & (g + 1 < NGRP))
        def _():
            tgt_desc(1 - tslot).start()

        for d in rows_descs(cur):
            d.wait()

        @pl.when(u == 0)
        def _():
            tgt_desc(tslot).wait()

        @pl.when(c + 2 < NCHUNK)
        def _():
            for d in stage_idx_descs(cur, c + 2):
                d.start()

        @pl.when((u == 0) & (g >= 2))
        def _():
            for d in out_descs(oslot, g - 2):
                d.wait()

        @pl.when((u == 0) & (g + 1 < NGRP))
        def _():
            pltpu.sync_copy(tgt_idx_hbm.at[pl.ds(wbase + (g + 1) * NL, NL)],
                            tgt_idx_v.at[pl.ds((1 - tslot) * NL, NL)])

        def b_body(b, ts_vec):
            rbase = b * CTX

            def j_body(j, acc):
                r = rbase + j
                return tuple(acc[d] + ctx_rows[cur, r, pl.ds(NL * d, NL)]
                             for d in range(ND))

            acc = lax.fori_loop(
                0, CTX, j_body,
                tuple(jnp.zeros((NL,), jnp.float32) for _ in range(ND)))

            def dot(r):
                q = acc[0] * neg_rows[cur, r, pl.ds(0, NL)]
                for d in range(1, ND):
                    q = q + acc[d] * neg_rows[cur, r, pl.ds(NL * d, NL)]
                return q

            trow = u * CB + b
            p = acc[0] * tgt_rows[tslot, trow, pl.ds(0, NL)]
            for d in range(1, ND):
                p = p + acc[d] * tgt_rows[tslot, trow, pl.ds(NL * d, NL)]
            ts_vec = jnp.where(lanes == trow, _allsum(p, lanes), ts_vec)

            for gg in range(3):
                qs = [dot(rbase + gg * NL + k) for k in range(NL)]
                ns_gbuf[oslot, trow, pl.ds(gg * NL, NL)] = _reduce16(
                    qs, lanes, masks)

            s48 = _allsum(dot(rbase + 48), lanes)
            s49 = _allsum(dot(rbase + 49), lanes)
            tail = jnp.where(lanes == 0, s48,
                             jnp.where(lanes == 1, s49, 0.0))
            ns_gbuf[oslot, trow, pl.ds(48, NL)] = tail
            return ts_vec

        ts_vec = lax.fori_loop(0, CB, b_body, ts_vec)

        @pl.when(u == GSZ - 1)
        def _():
            ts_gbuf[pl.ds(oslot * NL, NL)] = ts_vec
            for d in out_descs(oslot, g):
                d.start()

        return jnp.where(u == GSZ - 1, jnp.zeros((NL,), jnp.float32), ts_vec)

    lax.fori_loop(0, NCHUNK, chunk_body, jnp.zeros((NL,), jnp.float32))

    for gg in (NGRP - 2, NGRP - 1):
        for d in out_descs(gg % 2, gg):
            d.wait()


_sc_call = functools.partial(
    pl.kernel,
    mesh=plsc.VectorSubcoreMesh(core_axis_name="c", subcore_axis_name="s"),
    out_type=[
        jax.ShapeDtypeStruct((B,), jnp.float32),
        jax.ShapeDtypeStruct((B, NSLOT), jnp.float32),
    ],
    scratch_types=[
        pltpu.VMEM((2 * NIDX,), jnp.int32),
        pltpu.VMEM((2 * NIDX,), jnp.int32),
        pltpu.VMEM((2 * NL,), jnp.int32),
        pltpu.VMEM((2, NIDX, DIM), jnp.float32),
        pltpu.VMEM((2, NIDX, DIM), jnp.float32),
        pltpu.VMEM((2, NL, DIM), jnp.float32),
        pltpu.VMEM((2, NL, NSLOT), jnp.float32),
        pltpu.VMEM((2 * NL,), jnp.float32),
        pltpu.SemaphoreType.DMA,
        pltpu.SemaphoreType.DMA,
        pltpu.SemaphoreType.DMA,
        pltpu.SemaphoreType.DMA,
        pltpu.SemaphoreType.DMA,
    ],
)(_sc_body)


def _loss_body(ts_ref, ns_ref, out_ref):
    ts = ts_ref[...]
    ns = ns_ref[...]

    def softplus(x):
        return jnp.maximum(x, 0.0) + jnp.log1p(jnp.exp(-jnp.abs(x)))

    t_term = jnp.sum(softplus(-ts)) / B
    mask = lax.broadcasted_iota(jnp.int32, ns.shape, 1) < NNEG
    n_term = jnp.sum(jnp.where(mask, softplus(ns), 0.0)) / (B * NNEG)
    out_ref[0, 0] = t_term + n_term


_loss_call = pl.pallas_call(
    _loss_body,
    out_shape=jax.ShapeDtypeStruct((1, 1), jnp.float32),
    out_specs=pl.BlockSpec(memory_space=pltpu.SMEM),
)


def kernel(context, target, negative_samples, emb_in, emb_out):
    ctx_flat = context.reshape(-1)
    neg_flat = negative_samples.reshape(-1)
    ts, ns = _sc_call(ctx_flat, target, neg_flat, emb_in, emb_out)
    loss = _loss_call(ts.reshape(B // DIM, DIM), ns)
    return loss[0, 0]


# speedup vs baseline: 2.7259x; 1.0027x over previous
import functools

import jax
import jax.numpy as jnp
from jax import lax
from jax.experimental import pallas as pl
from jax.experimental.pallas import tpu as pltpu
from jax.experimental.pallas import tpu_sc as plsc

VOCAB = 100000
DIM = 128
B = 16384
CTX = 50
NNEG = 50
NSLOT = 64

NC = 2
NS = 16
NW = NC * NS
BPW = B // NW
CB = 4
NCHUNK = BPW // CB
NIDX = CB * CTX
GSZ = 4
NGRP = NCHUNK // GSZ
SUP = 4
SNIDX = SUP * NIDX
NSUP = NCHUNK // SUP
NL = 16
ND = DIM // NL

_GATHER_DNUMS = lax.GatherDimensionNumbers(
    offset_dims=(), collapsed_slice_dims=(0,), start_index_map=(0,))


def _lane_gather(v, idx):
    return lax.gather(v, idx[:, None], _GATHER_DNUMS, (1,),
                      mode=lax.GatherScatterMode.PROMISE_IN_BOUNDS)


def _allsum(v, lanes):
    for sh in (8, 4, 2, 1):
        v = v + _lane_gather(v, lanes ^ sh)
    return v


def _reduce16(vs, lanes, masks):
    sh = 1
    for m in masks:
        half = []
        for i in range(len(vs) // 2):
            a, b = vs[2 * i], vs[2 * i + 1]
            half.append(jnp.where(m, a + _lane_gather(a, lanes ^ sh),
                                  b + _lane_gather(b, lanes ^ sh)))
        vs = half
        sh *= 2
    return vs[0]


def _sc_body(ctx_idx_hbm, tgt_idx_hbm, neg_idx_hbm, emb_in_hbm, emb_out_hbm,
             ts_out, ns_out,
             ctx_idx_v, neg_idx_v, tgt_idx_v,
             ctx_rows, neg_rows, tgt_rows,
             ns_gbuf, ts_gbuf, sem_c, sem_n, sem_t, sem_i, sem_o):
    cid = lax.axis_index("c")
    sid = lax.axis_index("s")
    wid = sid * NC + cid
    wbase = wid * BPW
    lanes = lax.iota(jnp.int32, NL)
    masks = [(lanes & sh) == 0 for sh in (1, 2, 4, 8)]

    def stage_idx_descs(sp):
        base_b = wbase + sp * SUP * CB
        sslot = sp % 2
        return (
            pltpu.make_async_copy(
                ctx_idx_hbm.at[pl.ds(base_b * CTX, SNIDX)],
                ctx_idx_v.at[pl.ds(sslot * SNIDX, SNIDX)], sem_i),
            pltpu.make_async_copy(
                neg_idx_hbm.at[pl.ds(base_b * NNEG, SNIDX)],
                neg_idx_v.at[pl.ds(sslot * SNIDX, SNIDX)], sem_i),
        )

    def rows_descs(c):
        slot = c % 2
        off = ((c // SUP) % 2) * SNIDX + (c % SUP) * NIDX
        return (
            pltpu.make_async_copy(
                emb_in_hbm.at[ctx_idx_v.at[pl.ds(off, NIDX)]],
                ctx_rows.at[slot], sem_c),
            pltpu.make_async_copy(
                emb_out_hbm.at[neg_idx_v.at[pl.ds(off, NIDX)]],
                neg_rows.at[slot], sem_n),
        )

    def out_descs(oslot, g):
        return (
            pltpu.make_async_copy(
                ns_gbuf.at[oslot],
                ns_out.at[pl.ds(wbase + g * NL, NL), :], sem_o),
            pltpu.make_async_copy(
                ts_gbuf.at[pl.ds(oslot * NL, NL)],
                ts_out.at[pl.ds(wbase + g * NL, NL)], sem_o),
        )

    def tgt_desc(slot):
        return pltpu.make_async_copy(
            emb_out_hbm.at[tgt_idx_v.at[pl.ds(slot * NL, NL)]],
            tgt_rows.at[slot], sem_t)

    pltpu.sync_copy(ctx_idx_hbm.at[pl.ds(wbase * CTX, SNIDX)],
                    ctx_idx_v.at[pl.ds(0, SNIDX)])
    pltpu.sync_copy(neg_idx_hbm.at[pl.ds(wbase * NNEG, SNIDX)],
                    neg_idx_v.at[pl.ds(0, SNIDX)])
    pltpu.sync_copy(tgt_idx_hbm.at[pl.ds(wbase, NL)], tgt_idx_v.at[pl.ds(0, NL)])
    for d in rows_descs(0):
        d.start()
    tgt_desc(0).start()

    def chunk_body(c, ts_vec):
        cur = c % 2
        nxt = 1 - cur
        g = c // GSZ
        u = c % GSZ
        tslot = g % 2
        oslot = tslot

        sp = c // SUP
        u4 = c % SUP

        @pl.when(c + 1 < NCHUNK)
        def _():
            @pl.when(u4 == SUP - 1)
            def _():
                for d in stage_idx_descs(sp + 1):
                    d.wait()

            for d in rows_descs(c + 1):
                d.start()

        @pl.when((u == GSZ - 1) & (g + 1 < NGRP))
        def _():
            tgt_desc(1 - tslot).start()

        for d in rows_descs(c):
            d.wait()

        @pl.when(u == 0)
        def _():
            tgt_desc(tslot).wait()

        @pl.when((u4 == 0) & (sp + 1 < NSUP))
        def _():
            for d in stage_idx_descs(sp + 1):
                d.start()

        @pl.when((u == 0) & (g >= 2))
        def _():
            for d in out_descs(oslot, g - 2):
                d.wait()

        @pl.when((u == 0) & (g + 1 < NGRP))
        def _():
            pltpu.sync_copy(tgt_idx_hbm.at[pl.ds(wbase + (g + 1) * NL, NL)],
                            tgt_idx_v.at[pl.ds((1 - tslot) * NL, NL)])

        def b_body(b, ts_vec):
            rbase = b * CTX

            def j_body(j, acc):
                r = rbase + j
                return tuple(acc[d] + ctx_rows[cur, r, pl.ds(NL * d, NL)]
                             for d in range(ND))

            acc = lax.fori_loop(
                0, CTX, j_body,
                tuple(jnp.zeros((NL,), jnp.float32) for _ in range(ND)))

            def dot(r):
                q = acc[0] * neg_rows[cur, r, pl.ds(0, NL)]
                for d in range(1, ND):
                    q = q + acc[d] * neg_rows[cur, r, pl.ds(NL * d, NL)]
                return q

            trow = u * CB + b
            p = acc[0] * tgt_rows[tslot, trow, pl.ds(0, NL)]
            for d in range(1, ND):
                p = p + acc[d] * tgt_rows[tslot, trow, pl.ds(NL * d, NL)]
            ts_vec = jnp.where(lanes == trow, _allsum(p, lanes), ts_vec)

            for gg in range(3):
                qs = [dot(rbase + gg * NL + k) for k in range(NL)]
                ns_gbuf[oslot, trow, pl.ds(gg * NL, NL)] = _reduce16(
                    qs, lanes, masks)

            s48 = _allsum(dot(rbase + 48), lanes)
            s49 = _allsum(dot(rbase + 49), lanes)
            tail = jnp.where(lanes == 0, s48,
                             jnp.where(lanes == 1, s49, 0.0))
            ns_gbuf[oslot, trow, pl.ds(48, NL)] = tail
            return ts_vec

        ts_vec = lax.fori_loop(0, CB, b_body, ts_vec)

        @pl.when(u == GSZ - 1)
        def _():
            ts_gbuf[pl.ds(oslot * NL, NL)] = ts_vec
            for d in out_descs(oslot, g):
                d.start()

        return jnp.where(u == GSZ - 1, jnp.zeros((NL,), jnp.float32), ts_vec)

    lax.fori_loop(0, NCHUNK, chunk_body, jnp.zeros((NL,), jnp.float32))

    for gg in (NGRP - 2, NGRP - 1):
        for d in out_descs(gg % 2, gg):
            d.wait()


_sc_call = functools.partial(
    pl.kernel,
    mesh=plsc.VectorSubcoreMesh(core_axis_name="c", subcore_axis_name="s"),
    out_type=[
        jax.ShapeDtypeStruct((B,), jnp.float32),
        jax.ShapeDtypeStruct((B, NSLOT), jnp.float32),
    ],
    scratch_types=[
        pltpu.VMEM((2 * SNIDX,), jnp.int32),
        pltpu.VMEM((2 * SNIDX,), jnp.int32),
        pltpu.VMEM((2 * NL,), jnp.int32),
        pltpu.VMEM((2, NIDX, DIM), jnp.float32),
        pltpu.VMEM((2, NIDX, DIM), jnp.float32),
        pltpu.VMEM((2, NL, DIM), jnp.float32),
        pltpu.VMEM((2, NL, NSLOT), jnp.float32),
        pltpu.VMEM((2 * NL,), jnp.float32),
        pltpu.SemaphoreType.DMA,
        pltpu.SemaphoreType.DMA,
        pltpu.SemaphoreType.DMA,
        pltpu.SemaphoreType.DMA,
        pltpu.SemaphoreType.DMA,
    ],
)(_sc_body)


def _loss_body(ts_ref, ns_ref, out_ref):
    ts = ts_ref[...]
    ns = ns_ref[...]

    def softplus(x):
        return jnp.maximum(x, 0.0) + jnp.log1p(jnp.exp(-jnp.abs(x)))

    t_term = jnp.sum(softplus(-ts)) / B
    mask = lax.broadcasted_iota(jnp.int32, ns.shape, 1) < NNEG
    n_term = jnp.sum(jnp.where(mask, softplus(ns), 0.0)) / (B * NNEG)
    out_ref[0, 0] = t_term + n_term


_loss_call = pl.pallas_call(
    _loss_body,
    out_shape=jax.ShapeDtypeStruct((1, 1), jnp.float32),
    out_specs=pl.BlockSpec(memory_space=pltpu.SMEM),
)


def kernel(context, target, negative_samples, emb_in, emb_out):
    ctx_flat = context.reshape(-1)
    neg_flat = negative_samples.reshape(-1)
    ts, ns = _sc_call(ctx_flat, target, neg_flat, emb_in, emb_out)
    loss = _loss_call(ts.reshape(B // DIM, DIM), ns)
    return loss[0, 0]
